# baseline (speedup 1.0000x reference)
import functools

import jax
import jax.numpy as jnp
from jax import lax
from jax.experimental import pallas as pl
from jax.experimental.pallas import tpu as pltpu
from jax.experimental.pallas import tpu_sc as plsc

N = 100000
E = 1600000
HID = 32
NR = N // 4
BLK = 1000
NB = NR // BLK
GBLK = 4 * BLK

SC_NC = 2
SC_NS = 16
SUB = 128
JJ = 4
CHUNK = JJ * SUB
KK = 196
EPT = KK * CHUNK
EPAD = SC_NS * EPT
ACC_R = 100096
ZCH = ACC_R // SC_NS
DCH = N // SC_NS


@functools.lru_cache(maxsize=None)
def _make_sc_scatter(halves):
    mesh = plsc.VectorSubcoreMesh(core_axis_name="c", subcore_axis_name="s",
                                  num_cores=SC_NC, num_subcores=SC_NS)

    @functools.partial(
        pl.kernel,
        out_type=jax.ShapeDtypeStruct((N, 2, 16), jnp.float32),
        mesh=mesh,
        scratch_types=[
            pltpu.VMEM_SHARED((ACC_R, 16), jnp.float32),
            [pltpu.VMEM((JJ, SUB), jnp.int32) for _ in range(2)],
            [pltpu.VMEM((JJ, SUB), jnp.int32) for _ in range(2)],
            [pltpu.VMEM((CHUNK, 16), jnp.float32) for _ in range(2)],
            pltpu.SemaphoreType.DMA,
            [pltpu.SemaphoreType.DMA for _ in range(2)],
        ],
        compiler_params=pltpu.CompilerParams(use_tc_tiling_on_sc=False),
    )
    def sc_scatter(h2, src2, dst2, zrows, out, acc, idx_v, dst_v, rows_v,
                   gsem, ssem):
        c = lax.axis_index("c")
        s = lax.axis_index("s")

        pltpu.sync_copy(zrows, acc.at[pl.ds(s * ZCH, ZCH)])
        plsc.subcore_barrier()

        hview = h2.at[pl.ds(0 if halves == 1 else c, 2 * N - 1)]

        def one_chunk(t, p, first):
            @pl.when(jnp.logical_not(first))
            def _():
                pltpu.make_async_copy(
                    rows_v[p], acc.at[pl.ds(0, CHUNK)], ssem[p]).wait()
            row0 = t * JJ
            pltpu.sync_copy(src2.at[pl.ds(row0, JJ)], idx_v[p])
            pltpu.sync_copy(dst2.at[pl.ds(row0, JJ)], dst_v[p])
            for j in range(JJ):
                pltpu.async_copy(hview.at[idx_v[p].at[j]],
                                 rows_v[p].at[pl.ds(j * SUB, SUB)], gsem)
            pltpu.make_async_copy(h2.at[pl.ds(0, CHUNK)], rows_v[p],
                                  gsem).wait()
            for j in range(JJ):
                pltpu.async_copy(rows_v[p].at[pl.ds(j * SUB, SUB)],
                                 acc.at[dst_v[p].at[j]], ssem[p], add=True)

        def make_body(base):
            def body(k2, _):
                one_chunk(base + 2 * k2, 0, k2 == 0)
                one_chunk(base + 2 * k2 + 1, 1, k2 == 0)
                return 0
            return body

        def drain_all():
            for p in range(2):
                pltpu.make_async_copy(
                    rows_v[p], acc.at[pl.ds(0, CHUNK)], ssem[p]).wait()

        if halves == 1:
            w = c * SC_NS + s
            lax.fori_loop(0, KK // 4, make_body(w * (KK // 2)), 0)
            drain_all()
        else:
            lax.fori_loop(0, KK // 2, make_body(s * KK), 0)
            drain_all()

        plsc.subcore_barrier()
        pltpu.sync_copy(acc.at[pl.ds(s * DCH, DCH)],
                        out.at[pl.ds(s * DCH, DCH), c])

    return sc_scatter


def _form_m(a_ref, h_ref, agg_ref, l1):
    agg = agg_ref[...]
    if l1:
        pieces = []
        for g in range(4):
            p0 = agg[:, 32 * g:32 * g + 16]
            p1 = agg[:, 32 * g + 16:32 * g + 32]
            pieces += [p0 + p1, jnp.zeros_like(p1)]
        agg = jnp.concatenate(pieces, axis=1)
    return a_ref[0, 0] * h_ref[...] + agg


def _stats_body(l1, a_ref, h_ref, agg_ref, sum_out, m2_out, acc_sum, acc_m2):
    i = pl.program_id(0)

    @pl.when(i == 0)
    def _():
        acc_sum[...] = jnp.zeros_like(acc_sum)
        acc_m2[...] = jnp.zeros_like(acc_m2)

    m = _form_m(a_ref, h_ref, agg_ref, l1)
    acc_sum[...] += jnp.sum(m, axis=0, keepdims=True)
    acc_m2[...] += lax.dot_general(m, m, (((0,), (0,)), ((), ())),
                                   preferred_element_type=jnp.float32)

    @pl.when(i == NB - 1)
    def _():
        sum_out[...] = acc_sum[...]
        m2_out[...] = acc_m2[...]


def _stats_pass(a, h, agg, l1=False):
    return pl.pallas_call(
        functools.partial(_stats_body, l1),
        grid=(NB,),
        in_specs=[
            pl.BlockSpec(memory_space=pltpu.SMEM),
            pl.BlockSpec((BLK, 128), lambda i: (i, 0)),
            pl.BlockSpec((BLK, 128), lambda i: (i, 0)),
        ],
        out_specs=[
            pl.BlockSpec((1, 128), lambda i: (0, 0)),
            pl.BlockSpec((128, 128), lambda i: (0, 0)),
        ],
        out_shape=[
            jax.ShapeDtypeStruct((1, 128), jnp.float32),
            jax.ShapeDtypeStruct((128, 128), jnp.float32),
        ],
        scratch_shapes=[
            pltpu.VMEM((1, 128), jnp.float32),
            pltpu.VMEM((128, 128), jnp.float32),
        ],
    )(a, h, agg)


def _mlp_body(l1, a_ref, h_ref, agg_ref, w1_ref, b1_ref, w2_ref, b2_ref,
              y2_out, s_out, ss_out, acc_s, acc_ss):
    i = pl.program_id(0)

    @pl.when(i == 0)
    def _():
        acc_s[...] = jnp.zeros_like(acc_s)
        acc_ss[...] = jnp.zeros_like(acc_ss)

    m = _form_m(a_ref, h_ref, agg_ref, l1)
    y1 = jnp.dot(m, w1_ref[...], preferred_element_type=jnp.float32) + b1_ref[...]
    z = jnp.maximum(y1, 0.0)
    y2 = jnp.dot(z, w2_ref[...], preferred_element_type=jnp.float32) + b2_ref[...]
    y2_out[...] = y2
    acc_s[...] += jnp.sum(y2, axis=0, keepdims=True)
    acc_ss[...] += jnp.sum(y2 * y2, axis=0, keepdims=True)

    @pl.when(i == NB - 1)
    def _():
        s_out[...] = acc_s[...]
        ss_out[...] = acc_ss[...]


def _mlp_pass(a, h, agg, w1p, b1p, w2, b2, l1=False):
    return pl.pallas_call(
        functools.partial(_mlp_body, l1),
        grid=(NB,),
        in_specs=[
            pl.BlockSpec(memory_space=pltpu.SMEM),
            pl.BlockSpec((BLK, 128), lambda i: (i, 0)),
            pl.BlockSpec((BLK, 128), lambda i: (i, 0)),
            pl.BlockSpec((128, 128), lambda i: (0, 0)),
            pl.BlockSpec((1, 128), lambda i: (0, 0)),
            pl.BlockSpec((128, 128), lambda i: (0, 0)),
            pl.BlockSpec((1, 128), lambda i: (0, 0)),
        ],
        out_specs=[
            pl.BlockSpec((BLK, 128), lambda i: (i, 0)),
            pl.BlockSpec((1, 128), lambda i: (0, 0)),
            pl.BlockSpec((1, 128), lambda i: (0, 0)),
        ],
        out_shape=[
            jax.ShapeDtypeStruct((NR, 128), jnp.float32),
            jax.ShapeDtypeStruct((1, 128), jnp.float32),
            jax.ShapeDtypeStruct((1, 128), jnp.float32),
        ],
        scratch_shapes=[
            pltpu.VMEM((1, 128), jnp.float32),
            pltpu.VMEM((1, 128), jnp.float32),
        ],
    )(a, h, agg, w1p, b1p, w2, b2)


def _norm_body(y2_ref, sc_ref, bi_ref, wf_ref, accin_ref, h_out, acc_out):
    y2 = y2_ref[...]
    h = jnp.maximum(y2 * sc_ref[...] + bi_ref[...], 0.0)
    h_out[...] = h
    acc_out[...] = accin_ref[...] + jnp.dot(
        h, wf_ref[...], preferred_element_type=jnp.float32)


def _norm_pass(y2, scale, bias, wfp, accin):
    return pl.pallas_call(
        _norm_body,
        grid=(NB,),
        in_specs=[
            pl.BlockSpec((BLK, 128), lambda i: (i, 0)),
            pl.BlockSpec((1, 128), lambda i: (0, 0)),
            pl.BlockSpec((1, 128), lambda i: (0, 0)),
            pl.BlockSpec((128, 128), lambda i: (0, 0)),
            pl.BlockSpec((BLK, 128), lambda i: (i, 0)),
        ],
        out_specs=[
            pl.BlockSpec((BLK, 128), lambda i: (i, 0)),
            pl.BlockSpec((BLK, 128), lambda i: (i, 0)),
        ],
        out_shape=[
            jax.ShapeDtypeStruct((NR, 128), jnp.float32),
            jax.ShapeDtypeStruct((NR, 128), jnp.float32),
        ],
        input_output_aliases={4: 1},
    )(y2, scale, bias, wfp, accin)


def _build_body(t_ref, x0, x1, x2, x3, h_out):
    t = t_ref[0, 0]
    cols = []
    for xg in (x0, x1, x2, x3):
        xv = xg[...]
        pad = jnp.zeros((BLK, 20), jnp.float32)
        grp = jnp.concatenate([xv, pad], axis=1)
        col = lax.broadcasted_iota(jnp.int32, grp.shape, 1)
        grp = jnp.where(col == 12, t, grp)
        cols.append(grp)
    h_out[...] = jnp.concatenate(cols, axis=1)


def _build_pass(t, xs4):
    return pl.pallas_call(
        _build_body,
        grid=(NB,),
        in_specs=[pl.BlockSpec(memory_space=pltpu.SMEM)] + [
            pl.BlockSpec((BLK, 12), lambda i: (i, 0)) for _ in range(4)
        ],
        out_specs=pl.BlockSpec((BLK, 128), lambda i: (i, 0)),
        out_shape=jax.ShapeDtypeStruct((NR, 128), jnp.float32),
    )(t, *xs4)


def _final_body(ond_ref, oed_ref, bf_ref, acc_ref,
                x0, x1, x2, x3, m0, m1, m2, m3, o0, o1, o2, o3):
    acc = acc_ref[...]
    bf = bf_ref[...]
    xs = (x0, x1, x2, x3)
    ms = (m0, m1, m2, m3)
    outs = (o0, o1, o2, o3)
    for g in range(4):
        newx = acc[:, 32 * g:32 * g + 12] + bf
        x = xs[g][...]
        mc = ms[g][...]
        col = lax.broadcasted_iota(jnp.int32, x.shape, 1)
        incol1 = (col >= 1) & (col < ond_ref[0, 0] + 1)
        incol2 = (col >= 1) & (col < oed_ref[0, 0] + 1)
        nm = ((mc & 1) > 0) & incol1
        em = ((mc & 2) > 0) & incol2
        out = jnp.where(nm, newx, x)
        outs[g][...] = jnp.where(em, newx, out)


def _final_pass(ond, oed, bf, acc, xs4, ms4):
    outs = pl.pallas_call(
        _final_body,
        grid=(NB,),
        in_specs=[
            pl.BlockSpec(memory_space=pltpu.SMEM),
            pl.BlockSpec(memory_space=pltpu.SMEM),
            pl.BlockSpec((1, 12), lambda i: (0, 0)),
            pl.BlockSpec((BLK, 128), lambda i: (i, 0)),
        ] + [pl.BlockSpec((BLK, 12), lambda i: (i, 0)) for _ in range(4)]
          + [pl.BlockSpec((BLK, 1), lambda i: (i, 0)) for _ in range(4)],
        out_specs=[pl.BlockSpec((BLK, 12), lambda i: (i, 0)) for _ in range(4)],
        out_shape=[jax.ShapeDtypeStruct((NR, 12), jnp.float32)] * 4,
    )(ond, oed, bf, acc, *xs4, *ms4)
    return jnp.concatenate(outs, axis=0)


def _fold_bn1(sum128, m2128, w1, b1, g1, be1):
    sum_m = sum128.reshape(4, HID).sum(axis=0, keepdims=True)
    m2 = (m2128[0:32, 0:32] + m2128[32:64, 32:64]
          + m2128[64:96, 64:96] + m2128[96:128, 96:128])
    mu_m = sum_m / N
    cov = m2 / N - mu_m.T @ mu_m
    mu1 = mu_m @ w1 + b1
    var1 = jnp.sum(w1 * (cov @ w1), axis=0, keepdims=True)
    scale = g1 / jnp.sqrt(var1 + 1e-5)
    w1p = w1 * scale
    b1p = (b1 - mu1) * scale + be1
    return w1p, b1p


def _fold_bn2(s128, ss128, gn, bn):
    s = s128.reshape(4, HID).sum(axis=0, keepdims=True)
    ss = ss128.reshape(4, HID).sum(axis=0, keepdims=True)
    mu = s / N
    var = ss / N - mu * mu
    scale = gn / jnp.sqrt(var + 1e-5)
    bias = bn - mu * scale
    return scale, bias


def _blockdiag(w):
    return jnp.kron(jnp.eye(4, dtype=jnp.float32), w)


def kernel(x, t, edge_index, node_mask, edge_mask, ond, oed, params):
    layer_params, (wf, bf) = params
    src = edge_index[0]
    dst = edge_index[1]

    npad = EPAD - E
    qsrc = 2 * ((src % NR) * 4 + src // NR)
    qdst = (dst % NR) * 4 + dst // NR
    src2 = jnp.concatenate([qsrc, jnp.zeros((npad,), jnp.int32)]
                           ).reshape(-1, SUB)
    dst2 = jnp.concatenate([qdst, jnp.full((npad,), N, jnp.int32)]
                           ).reshape(-1, SUB)
    zrows = jnp.zeros((ZCH, 16), jnp.float32)

    xs4 = [x[g * NR:(g + 1) * NR] for g in range(4)]
    h = _build_pass(t.reshape(1, 1), xs4)

    acc = jnp.zeros((NR, 128), jnp.float32)
    for l, (eps, w1, b1, g1, be1, w2, b2, gn, bn) in enumerate(layer_params):
        if l == 0:
            w1 = jnp.concatenate([w1, jnp.zeros((HID - 13, HID), jnp.float32)], 0)
        a = (1.0 + eps).reshape(1, 1)
        sc_fn = _make_sc_scatter(1 if l == 0 else 2)
        agg = sc_fn(h.reshape(2 * N, 16), src2, dst2, zrows).reshape(NR, 128)
        sum128, m2128 = _stats_pass(a, h, agg, l1=(l == 0))
        w1p, b1p = _fold_bn1(sum128, m2128, w1, b1.reshape(1, -1),
                             g1.reshape(1, -1), be1.reshape(1, -1))
        y2, s128, ss128 = _mlp_pass(a, h, agg, _blockdiag(w1p),
                                    jnp.tile(b1p, (1, 4)), _blockdiag(w2),
                                    jnp.tile(b2.reshape(1, -1), (1, 4)),
                                    l1=(l == 0))
        scale2, bias2 = _fold_bn2(s128, ss128, gn.reshape(1, -1),
                                  bn.reshape(1, -1))
        wfp = jnp.pad(wf[l * HID:(l + 1) * HID], ((0, 0), (0, 20)))
        h, acc = _norm_pass(y2, jnp.tile(scale2, (1, 4)),
                            jnp.tile(bias2, (1, 4)), _blockdiag(wfp), acc)

    mcode = (node_mask.astype(jnp.int32)
             + 2 * edge_mask.astype(jnp.int32)).reshape(N, 1)
    ms4 = [mcode[g * NR:(g + 1) * NR] for g in range(4)]
    ondp = jnp.asarray(ond, jnp.int32).reshape(1, 1)
    oedp = jnp.asarray(oed, jnp.int32).reshape(1, 1)
    return _final_pass(ondp, oedp, bf.reshape(1, -1), acc, xs4, ms4)


# speedup vs baseline: 1.0558x; 1.0558x over previous
import functools

import jax
import jax.numpy as jnp
from jax import lax
from jax.experimental import pallas as pl
from jax.experimental.pallas import tpu as pltpu
from jax.experimental.pallas import tpu_sc as plsc

N = 100000
E = 1600000
HID = 32
NR = N // 4
BLK = 1000
NB = NR // BLK
GBLK = 4 * BLK

SC_NC = 2
SC_NS = 16
SUB = 128
JJ = 8
CHUNK = JJ * SUB
KK = 98
EPT = KK * CHUNK
EPAD = SC_NS * EPT
ACC_R = 100096
ZCH = ACC_R // SC_NS
DCH = N // SC_NS


@functools.lru_cache(maxsize=None)
def _make_sc_scatter(halves):
    mesh = plsc.VectorSubcoreMesh(core_axis_name="c", subcore_axis_name="s",
                                  num_cores=SC_NC, num_subcores=SC_NS)

    @functools.partial(
        pl.kernel,
        out_type=jax.ShapeDtypeStruct((N, 2, 16), jnp.float32),
        mesh=mesh,
        scratch_types=[
            pltpu.VMEM_SHARED((ACC_R, 16), jnp.float32),
            pltpu.VMEM((JJ, SUB), jnp.int32),
            pltpu.VMEM((JJ, SUB), jnp.int32),
            pltpu.VMEM((CHUNK, 16), jnp.float32),
            pltpu.SemaphoreType.DMA,
        ],
        compiler_params=pltpu.CompilerParams(use_tc_tiling_on_sc=False),
    )
    def sc_scatter(h2, src2, dst2, zrows, out, acc, idx_v, dst_v, rows_v,
                   gsem):
        c = lax.axis_index("c")
        s = lax.axis_index("s")

        pltpu.sync_copy(zrows, acc.at[pl.ds(s * ZCH, ZCH)])
        plsc.subcore_barrier()

        hview = h2.at[pl.ds(0 if halves == 1 else c, 2 * N - 1)]

        def make_chunk_body(base):
            def chunk_body(k, _):
                row0 = base + k * JJ
                pltpu.sync_copy(src2.at[pl.ds(row0, JJ)], idx_v)
                pltpu.sync_copy(dst2.at[pl.ds(row0, JJ)], dst_v)
                for j in range(JJ):
                    pltpu.async_copy(hview.at[idx_v.at[j]],
                                     rows_v.at[pl.ds(j * SUB, SUB)], gsem)
                pltpu.make_async_copy(h2.at[pl.ds(0, CHUNK)], rows_v,
                                      gsem).wait()
                for j in range(JJ):
                    pltpu.sync_copy(rows_v.at[pl.ds(j * SUB, SUB)],
                                    acc.at[dst_v.at[j]], add=True)
                return 0
            return chunk_body

        if halves == 1:
            w = c * SC_NS + s
            lax.fori_loop(0, KK // 2,
                          make_chunk_body(w * (KK // 2 * JJ)), 0)
        else:
            lax.fori_loop(0, KK, make_chunk_body(s * (KK * JJ)), 0)

        plsc.subcore_barrier()
        pltpu.sync_copy(acc.at[pl.ds(s * DCH, DCH)],
                        out.at[pl.ds(s * DCH, DCH), c])

    return sc_scatter


def _form_m(a_ref, h_ref, agg_ref, l1):
    agg = agg_ref[...]
    if l1:
        pieces = []
        for g in range(4):
            p0 = agg[:, 32 * g:32 * g + 16]
            p1 = agg[:, 32 * g + 16:32 * g + 32]
            pieces += [p0 + p1, jnp.zeros_like(p1)]
        agg = jnp.concatenate(pieces, axis=1)
    return a_ref[0, 0] * h_ref[...] + agg


def _stats_body(l1, a_ref, h_ref, agg_ref, sum_out, m2_out, acc_sum, acc_m2):
    i = pl.program_id(0)

    @pl.when(i == 0)
    def _():
        acc_sum[...] = jnp.zeros_like(acc_sum)
        acc_m2[...] = jnp.zeros_like(acc_m2)

    m = _form_m(a_ref, h_ref, agg_ref, l1)
    acc_sum[...] += jnp.sum(m, axis=0, keepdims=True)
    acc_m2[...] += lax.dot_general(m, m, (((0,), (0,)), ((), ())),
                                   preferred_element_type=jnp.float32)

    @pl.when(i == NB - 1)
    def _():
        sum_out[...] = acc_sum[...]
        m2_out[...] = acc_m2[...]


def _stats_pass(a, h, agg, l1=False):
    return pl.pallas_call(
        functools.partial(_stats_body, l1),
        grid=(NB,),
        in_specs=[
            pl.BlockSpec(memory_space=pltpu.SMEM),
            pl.BlockSpec((BLK, 128), lambda i: (i, 0)),
            pl.BlockSpec((BLK, 128), lambda i: (i, 0)),
        ],
        out_specs=[
            pl.BlockSpec((1, 128), lambda i: (0, 0)),
            pl.BlockSpec((128, 128), lambda i: (0, 0)),
        ],
        out_shape=[
            jax.ShapeDtypeStruct((1, 128), jnp.float32),
            jax.ShapeDtypeStruct((128, 128), jnp.float32),
        ],
        scratch_shapes=[
            pltpu.VMEM((1, 128), jnp.float32),
            pltpu.VMEM((128, 128), jnp.float32),
        ],
    )(a, h, agg)


def _mlp_body(l1, a_ref, h_ref, agg_ref, w1_ref, b1_ref, w2_ref, b2_ref,
              y2_out, s_out, ss_out, acc_s, acc_ss):
    i = pl.program_id(0)

    @pl.when(i == 0)
    def _():
        acc_s[...] = jnp.zeros_like(acc_s)
        acc_ss[...] = jnp.zeros_like(acc_ss)

    m = _form_m(a_ref, h_ref, agg_ref, l1)
    y1 = jnp.dot(m, w1_ref[...], preferred_element_type=jnp.float32) + b1_ref[...]
    z = jnp.maximum(y1, 0.0)
    y2 = jnp.dot(z, w2_ref[...], preferred_element_type=jnp.float32) + b2_ref[...]
    y2_out[...] = y2
    acc_s[...] += jnp.sum(y2, axis=0, keepdims=True)
    acc_ss[...] += jnp.sum(y2 * y2, axis=0, keepdims=True)

    @pl.when(i == NB - 1)
    def _():
        s_out[...] = acc_s[...]
        ss_out[...] = acc_ss[...]


def _mlp_pass(a, h, agg, w1p, b1p, w2, b2, l1=False):
    return pl.pallas_call(
        functools.partial(_mlp_body, l1),
        grid=(NB,),
        in_specs=[
            pl.BlockSpec(memory_space=pltpu.SMEM),
            pl.BlockSpec((BLK, 128), lambda i: (i, 0)),
            pl.BlockSpec((BLK, 128), lambda i: (i, 0)),
            pl.BlockSpec((128, 128), lambda i: (0, 0)),
            pl.BlockSpec((1, 128), lambda i: (0, 0)),
            pl.BlockSpec((128, 128), lambda i: (0, 0)),
            pl.BlockSpec((1, 128), lambda i: (0, 0)),
        ],
        out_specs=[
            pl.BlockSpec((BLK, 128), lambda i: (i, 0)),
            pl.BlockSpec((1, 128), lambda i: (0, 0)),
            pl.BlockSpec((1, 128), lambda i: (0, 0)),
        ],
        out_shape=[
            jax.ShapeDtypeStruct((NR, 128), jnp.float32),
            jax.ShapeDtypeStruct((1, 128), jnp.float32),
            jax.ShapeDtypeStruct((1, 128), jnp.float32),
        ],
        scratch_shapes=[
            pltpu.VMEM((1, 128), jnp.float32),
            pltpu.VMEM((1, 128), jnp.float32),
        ],
    )(a, h, agg, w1p, b1p, w2, b2)


def _norm_body(y2_ref, sc_ref, bi_ref, wf_ref, accin_ref, h_out, acc_out):
    y2 = y2_ref[...]
    h = jnp.maximum(y2 * sc_ref[...] + bi_ref[...], 0.0)
    h_out[...] = h
    acc_out[...] = accin_ref[...] + jnp.dot(
        h, wf_ref[...], preferred_element_type=jnp.float32)


def _norm_pass(y2, scale, bias, wfp, accin):
    return pl.pallas_call(
        _norm_body,
        grid=(NB,),
        in_specs=[
            pl.BlockSpec((BLK, 128), lambda i: (i, 0)),
            pl.BlockSpec((1, 128), lambda i: (0, 0)),
            pl.BlockSpec((1, 128), lambda i: (0, 0)),
            pl.BlockSpec((128, 128), lambda i: (0, 0)),
            pl.BlockSpec((BLK, 128), lambda i: (i, 0)),
        ],
        out_specs=[
            pl.BlockSpec((BLK, 128), lambda i: (i, 0)),
            pl.BlockSpec((BLK, 128), lambda i: (i, 0)),
        ],
        out_shape=[
            jax.ShapeDtypeStruct((NR, 128), jnp.float32),
            jax.ShapeDtypeStruct((NR, 128), jnp.float32),
        ],
        input_output_aliases={4: 1},
    )(y2, scale, bias, wfp, accin)


def _build_body(t_ref, x0, x1, x2, x3, h_out):
    t = t_ref[0, 0]
    cols = []
    for xg in (x0, x1, x2, x3):
        xv = xg[...]
        pad = jnp.zeros((BLK, 20), jnp.float32)
        grp = jnp.concatenate([xv, pad], axis=1)
        col = lax.broadcasted_iota(jnp.int32, grp.shape, 1)
        grp = jnp.where(col == 12, t, grp)
        cols.append(grp)
    h_out[...] = jnp.concatenate(cols, axis=1)


def _build_pass(t, xs4):
    return pl.pallas_call(
        _build_body,
        grid=(NB,),
        in_specs=[pl.BlockSpec(memory_space=pltpu.SMEM)] + [
            pl.BlockSpec((BLK, 12), lambda i: (i, 0)) for _ in range(4)
        ],
        out_specs=pl.BlockSpec((BLK, 128), lambda i: (i, 0)),
        out_shape=jax.ShapeDtypeStruct((NR, 128), jnp.float32),
    )(t, *xs4)


def _final_body(ond_ref, oed_ref, bf_ref, acc_ref,
                x0, x1, x2, x3, m0, m1, m2, m3, o0, o1, o2, o3):
    acc = acc_ref[...]
    bf = bf_ref[...]
    xs = (x0, x1, x2, x3)
    ms = (m0, m1, m2, m3)
    outs = (o0, o1, o2, o3)
    for g in range(4):
        newx = acc[:, 32 * g:32 * g + 12] + bf
        x = xs[g][...]
        mc = ms[g][...]
        col = lax.broadcasted_iota(jnp.int32, x.shape, 1)
        incol1 = (col >= 1) & (col < ond_ref[0, 0] + 1)
        incol2 = (col >= 1) & (col < oed_ref[0, 0] + 1)
        nm = ((mc & 1) > 0) & incol1
        em = ((mc & 2) > 0) & incol2
        out = jnp.where(nm, newx, x)
        outs[g][...] = jnp.where(em, newx, out)


def _final_pass(ond, oed, bf, acc, xs4, ms4):
    outs = pl.pallas_call(
        _final_body,
        grid=(NB,),
        in_specs=[
            pl.BlockSpec(memory_space=pltpu.SMEM),
            pl.BlockSpec(memory_space=pltpu.SMEM),
            pl.BlockSpec((1, 12), lambda i: (0, 0)),
            pl.BlockSpec((BLK, 128), lambda i: (i, 0)),
        ] + [pl.BlockSpec((BLK, 12), lambda i: (i, 0)) for _ in range(4)]
          + [pl.BlockSpec((BLK, 1), lambda i: (i, 0)) for _ in range(4)],
        out_specs=[pl.BlockSpec((BLK, 12), lambda i: (i, 0)) for _ in range(4)],
        out_shape=[jax.ShapeDtypeStruct((NR, 12), jnp.float32)] * 4,
    )(ond, oed, bf, acc, *xs4, *ms4)
    return jnp.concatenate(outs, axis=0)


def _fold_bn1(sum128, m2128, w1, b1, g1, be1):
    sum_m = sum128.reshape(4, HID).sum(axis=0, keepdims=True)
    m2 = (m2128[0:32, 0:32] + m2128[32:64, 32:64]
          + m2128[64:96, 64:96] + m2128[96:128, 96:128])
    mu_m = sum_m / N
    cov = m2 / N - mu_m.T @ mu_m
    mu1 = mu_m @ w1 + b1
    var1 = jnp.sum(w1 * (cov @ w1), axis=0, keepdims=True)
    scale = g1 / jnp.sqrt(var1 + 1e-5)
    w1p = w1 * scale
    b1p = (b1 - mu1) * scale + be1
    return w1p, b1p


def _fold_bn2(s128, ss128, gn, bn):
    s = s128.reshape(4, HID).sum(axis=0, keepdims=True)
    ss = ss128.reshape(4, HID).sum(axis=0, keepdims=True)
    mu = s / N
    var = ss / N - mu * mu
    scale = gn / jnp.sqrt(var + 1e-5)
    bias = bn - mu * scale
    return scale, bias


def _blockdiag(w):
    return jnp.kron(jnp.eye(4, dtype=jnp.float32), w)


def kernel(x, t, edge_index, node_mask, edge_mask, ond, oed, params):
    layer_params, (wf, bf) = params
    src = edge_index[0]
    dst = edge_index[1]

    npad = EPAD - E
    qsrc = 2 * ((src % NR) * 4 + src // NR)
    qdst = (dst % NR) * 4 + dst // NR
    src2 = jnp.concatenate([qsrc, jnp.zeros((npad,), jnp.int32)]
                           ).reshape(-1, SUB)
    dst2 = jnp.concatenate([qdst, jnp.full((npad,), N, jnp.int32)]
                           ).reshape(-1, SUB)
    zrows = jnp.zeros((ZCH, 16), jnp.float32)

    xs4 = [x[g * NR:(g + 1) * NR] for g in range(4)]
    h = _build_pass(t.reshape(1, 1), xs4)

    acc = jnp.zeros((NR, 128), jnp.float32)
    for l, (eps, w1, b1, g1, be1, w2, b2, gn, bn) in enumerate(layer_params):
        if l == 0:
            w1 = jnp.concatenate([w1, jnp.zeros((HID - 13, HID), jnp.float32)], 0)
        a = (1.0 + eps).reshape(1, 1)
        sc_fn = _make_sc_scatter(1 if l == 0 else 2)
        agg = sc_fn(h.reshape(2 * N, 16), src2, dst2, zrows).reshape(NR, 128)
        sum128, m2128 = _stats_pass(a, h, agg, l1=(l == 0))
        w1p, b1p = _fold_bn1(sum128, m2128, w1, b1.reshape(1, -1),
                             g1.reshape(1, -1), be1.reshape(1, -1))
        y2, s128, ss128 = _mlp_pass(a, h, agg, _blockdiag(w1p),
                                    jnp.tile(b1p, (1, 4)), _blockdiag(w2),
                                    jnp.tile(b2.reshape(1, -1), (1, 4)),
                                    l1=(l == 0))
        scale2, bias2 = _fold_bn2(s128, ss128, gn.reshape(1, -1),
                                  bn.reshape(1, -1))
        wfp = jnp.pad(wf[l * HID:(l + 1) * HID], ((0, 0), (0, 20)))
        h, acc = _norm_pass(y2, jnp.tile(scale2, (1, 4)),
                            jnp.tile(bias2, (1, 4)), _blockdiag(wfp), acc)

    mcode = (node_mask.astype(jnp.int32)
             + 2 * edge_mask.astype(jnp.int32)).reshape(N, 1)
    ms4 = [mcode[g * NR:(g + 1) * NR] for g in range(4)]
    ondp = jnp.asarray(ond, jnp.int32).reshape(1, 1)
    oedp = jnp.asarray(oed, jnp.int32).reshape(1, 1)
    return _final_pass(ondp, oedp, bf.reshape(1, -1), acc, xs4, ms4)


# speedup vs baseline: 1.1274x; 1.0679x over previous
import functools

import jax
import jax.numpy as jnp
from jax import lax
from jax.experimental import pallas as pl
from jax.experimental.pallas import tpu as pltpu
from jax.experimental.pallas import tpu_sc as plsc

N = 100000
E = 1600000
HID = 32
NR = N // 4
BLK = 1000
NB = NR // BLK
GBLK = 4 * BLK

SC_NC = 2
SC_NS = 16
SUB = 128
JJ = 8
CHUNK = JJ * SUB
KK = 98
EPT = KK * CHUNK
EPAD = SC_NS * EPT
ACC_R = 100096
ZCH = ACC_R // SC_NS
DCH = N // SC_NS


@functools.lru_cache(maxsize=None)
def _make_sc_scatter(halves):
    mesh = plsc.VectorSubcoreMesh(core_axis_name="c", subcore_axis_name="s",
                                  num_cores=SC_NC, num_subcores=SC_NS)

    @functools.partial(
        pl.kernel,
        out_type=jax.ShapeDtypeStruct((N, 2, 16), jnp.float32),
        mesh=mesh,
        scratch_types=[
            pltpu.VMEM_SHARED((ACC_R, 16), jnp.float32),
            pltpu.VMEM((JJ, SUB), jnp.int32),
            pltpu.VMEM((JJ, SUB), jnp.int32),
            pltpu.VMEM((CHUNK, 16), jnp.float32),
            pltpu.SemaphoreType.DMA,
        ],
        compiler_params=pltpu.CompilerParams(use_tc_tiling_on_sc=False),
    )
    def sc_scatter(h2, src2, dst2, zrows, out, acc, idx_v, dst_v, rows_v,
                   gsem):
        c = lax.axis_index("c")
        s = lax.axis_index("s")

        pltpu.sync_copy(zrows, acc.at[pl.ds(s * ZCH, ZCH)])
        plsc.subcore_barrier()

        hview = h2.at[pl.ds(0 if halves == 1 else c, 2 * N - 1)]

        def make_chunk_body(base):
            def chunk_body(k, _):
                row0 = base + k * JJ
                pltpu.sync_copy(src2.at[pl.ds(row0, JJ)], idx_v)
                pltpu.sync_copy(dst2.at[pl.ds(row0, JJ)], dst_v)
                descs = [
                    pltpu.async_copy(hview.at[idx_v.at[j]],
                                     rows_v.at[pl.ds(j * SUB, SUB)], gsem)
                    for j in range(JJ)
                ]
                for j in range(JJ):
                    descs[j].wait()
                    pltpu.sync_copy(rows_v.at[pl.ds(j * SUB, SUB)],
                                    acc.at[dst_v.at[j]], add=True)
                return 0
            return chunk_body

        if halves == 1:
            w = c * SC_NS + s
            lax.fori_loop(0, KK // 2,
                          make_chunk_body(w * (KK // 2 * JJ)), 0)
        else:
            lax.fori_loop(0, KK, make_chunk_body(s * (KK * JJ)), 0)

        plsc.subcore_barrier()
        pltpu.sync_copy(acc.at[pl.ds(s * DCH, DCH)],
                        out.at[pl.ds(s * DCH, DCH), c])

    return sc_scatter


def _form_m(a_ref, h_ref, agg_ref, l1):
    agg = agg_ref[...]
    if l1:
        pieces = []
        for g in range(4):
            p0 = agg[:, 32 * g:32 * g + 16]
            p1 = agg[:, 32 * g + 16:32 * g + 32]
            pieces += [p0 + p1, jnp.zeros_like(p1)]
        agg = jnp.concatenate(pieces, axis=1)
    return a_ref[0, 0] * h_ref[...] + agg


def _stats_body(l1, a_ref, h_ref, agg_ref, sum_out, m2_out, acc_sum, acc_m2):
    i = pl.program_id(0)

    @pl.when(i == 0)
    def _():
        acc_sum[...] = jnp.zeros_like(acc_sum)
        acc_m2[...] = jnp.zeros_like(acc_m2)

    m = _form_m(a_ref, h_ref, agg_ref, l1)
    acc_sum[...] += jnp.sum(m, axis=0, keepdims=True)
    acc_m2[...] += lax.dot_general(m, m, (((0,), (0,)), ((), ())),
                                   preferred_element_type=jnp.float32)

    @pl.when(i == NB - 1)
    def _():
        sum_out[...] = acc_sum[...]
        m2_out[...] = acc_m2[...]


def _stats_pass(a, h, agg, l1=False):
    return pl.pallas_call(
        functools.partial(_stats_body, l1),
        grid=(NB,),
        in_specs=[
            pl.BlockSpec(memory_space=pltpu.SMEM),
            pl.BlockSpec((BLK, 128), lambda i: (i, 0)),
            pl.BlockSpec((BLK, 128), lambda i: (i, 0)),
        ],
        out_specs=[
            pl.BlockSpec((1, 128), lambda i: (0, 0)),
            pl.BlockSpec((128, 128), lambda i: (0, 0)),
        ],
        out_shape=[
            jax.ShapeDtypeStruct((1, 128), jnp.float32),
            jax.ShapeDtypeStruct((128, 128), jnp.float32),
        ],
        scratch_shapes=[
            pltpu.VMEM((1, 128), jnp.float32),
            pltpu.VMEM((128, 128), jnp.float32),
        ],
    )(a, h, agg)


def _mlp_body(l1, a_ref, h_ref, agg_ref, w1_ref, b1_ref, w2_ref, b2_ref,
              y2_out, s_out, ss_out, acc_s, acc_ss):
    i = pl.program_id(0)

    @pl.when(i == 0)
    def _():
        acc_s[...] = jnp.zeros_like(acc_s)
        acc_ss[...] = jnp.zeros_like(acc_ss)

    m = _form_m(a_ref, h_ref, agg_ref, l1)
    y1 = jnp.dot(m, w1_ref[...], preferred_element_type=jnp.float32) + b1_ref[...]
    z = jnp.maximum(y1, 0.0)
    y2 = jnp.dot(z, w2_ref[...], preferred_element_type=jnp.float32) + b2_ref[...]
    y2_out[...] = y2
    acc_s[...] += jnp.sum(y2, axis=0, keepdims=True)
    acc_ss[...] += jnp.sum(y2 * y2, axis=0, keepdims=True)

    @pl.when(i == NB - 1)
    def _():
        s_out[...] = acc_s[...]
        ss_out[...] = acc_ss[...]


def _mlp_pass(a, h, agg, w1p, b1p, w2, b2, l1=False):
    return pl.pallas_call(
        functools.partial(_mlp_body, l1),
        grid=(NB,),
        in_specs=[
            pl.BlockSpec(memory_space=pltpu.SMEM),
            pl.BlockSpec((BLK, 128), lambda i: (i, 0)),
            pl.BlockSpec((BLK, 128), lambda i: (i, 0)),
            pl.BlockSpec((128, 128), lambda i: (0, 0)),
            pl.BlockSpec((1, 128), lambda i: (0, 0)),
            pl.BlockSpec((128, 128), lambda i: (0, 0)),
            pl.BlockSpec((1, 128), lambda i: (0, 0)),
        ],
        out_specs=[
            pl.BlockSpec((BLK, 128), lambda i: (i, 0)),
            pl.BlockSpec((1, 128), lambda i: (0, 0)),
            pl.BlockSpec((1, 128), lambda i: (0, 0)),
        ],
        out_shape=[
            jax.ShapeDtypeStruct((NR, 128), jnp.float32),
            jax.ShapeDtypeStruct((1, 128), jnp.float32),
            jax.ShapeDtypeStruct((1, 128), jnp.float32),
        ],
        scratch_shapes=[
            pltpu.VMEM((1, 128), jnp.float32),
            pltpu.VMEM((1, 128), jnp.float32),
        ],
    )(a, h, agg, w1p, b1p, w2, b2)


def _norm_body(y2_ref, sc_ref, bi_ref, wf_ref, accin_ref, h_out, acc_out):
    y2 = y2_ref[...]
    h = jnp.maximum(y2 * sc_ref[...] + bi_ref[...], 0.0)
    h_out[...] = h
    acc_out[...] = accin_ref[...] + jnp.dot(
        h, wf_ref[...], preferred_element_type=jnp.float32)


def _norm_pass(y2, scale, bias, wfp, accin):
    return pl.pallas_call(
        _norm_body,
        grid=(NB,),
        in_specs=[
            pl.BlockSpec((BLK, 128), lambda i: (i, 0)),
            pl.BlockSpec((1, 128), lambda i: (0, 0)),
            pl.BlockSpec((1, 128), lambda i: (0, 0)),
            pl.BlockSpec((128, 128), lambda i: (0, 0)),
            pl.BlockSpec((BLK, 128), lambda i: (i, 0)),
        ],
        out_specs=[
            pl.BlockSpec((BLK, 128), lambda i: (i, 0)),
            pl.BlockSpec((BLK, 128), lambda i: (i, 0)),
        ],
        out_shape=[
            jax.ShapeDtypeStruct((NR, 128), jnp.float32),
            jax.ShapeDtypeStruct((NR, 128), jnp.float32),
        ],
        input_output_aliases={4: 1},
    )(y2, scale, bias, wfp, accin)


def _build_body(t_ref, x0, x1, x2, x3, h_out):
    t = t_ref[0, 0]
    cols = []
    for xg in (x0, x1, x2, x3):
        xv = xg[...]
        pad = jnp.zeros((BLK, 20), jnp.float32)
        grp = jnp.concatenate([xv, pad], axis=1)
        col = lax.broadcasted_iota(jnp.int32, grp.shape, 1)
        grp = jnp.where(col == 12, t, grp)
        cols.append(grp)
    h_out[...] = jnp.concatenate(cols, axis=1)


def _build_pass(t, xs4):
    return pl.pallas_call(
        _build_body,
        grid=(NB,),
        in_specs=[pl.BlockSpec(memory_space=pltpu.SMEM)] + [
            pl.BlockSpec((BLK, 12), lambda i: (i, 0)) for _ in range(4)
        ],
        out_specs=pl.BlockSpec((BLK, 128), lambda i: (i, 0)),
        out_shape=jax.ShapeDtypeStruct((NR, 128), jnp.float32),
    )(t, *xs4)


def _final_body(ond_ref, oed_ref, bf_ref, acc_ref,
                x0, x1, x2, x3, m0, m1, m2, m3, o0, o1, o2, o3):
    acc = acc_ref[...]
    bf = bf_ref[...]
    xs = (x0, x1, x2, x3)
    ms = (m0, m1, m2, m3)
    outs = (o0, o1, o2, o3)
    for g in range(4):
        newx = acc[:, 32 * g:32 * g + 12] + bf
        x = xs[g][...]
        mc = ms[g][...]
        col = lax.broadcasted_iota(jnp.int32, x.shape, 1)
        incol1 = (col >= 1) & (col < ond_ref[0, 0] + 1)
        incol2 = (col >= 1) & (col < oed_ref[0, 0] + 1)
        nm = ((mc & 1) > 0) & incol1
        em = ((mc & 2) > 0) & incol2
        out = jnp.where(nm, newx, x)
        outs[g][...] = jnp.where(em, newx, out)


def _final_pass(ond, oed, bf, acc, xs4, ms4):
    outs = pl.pallas_call(
        _final_body,
        grid=(NB,),
        in_specs=[
            pl.BlockSpec(memory_space=pltpu.SMEM),
            pl.BlockSpec(memory_space=pltpu.SMEM),
            pl.BlockSpec((1, 12), lambda i: (0, 0)),
            pl.BlockSpec((BLK, 128), lambda i: (i, 0)),
        ] + [pl.BlockSpec((BLK, 12), lambda i: (i, 0)) for _ in range(4)]
          + [pl.BlockSpec((BLK, 1), lambda i: (i, 0)) for _ in range(4)],
        out_specs=[pl.BlockSpec((BLK, 12), lambda i: (i, 0)) for _ in range(4)],
        out_shape=[jax.ShapeDtypeStruct((NR, 12), jnp.float32)] * 4,
    )(ond, oed, bf, acc, *xs4, *ms4)
    return jnp.concatenate(outs, axis=0)


def _fold_bn1(sum128, m2128, w1, b1, g1, be1):
    sum_m = sum128.reshape(4, HID).sum(axis=0, keepdims=True)
    m2 = (m2128[0:32, 0:32] + m2128[32:64, 32:64]
          + m2128[64:96, 64:96] + m2128[96:128, 96:128])
    mu_m = sum_m / N
    cov = m2 / N - mu_m.T @ mu_m
    mu1 = mu_m @ w1 + b1
    var1 = jnp.sum(w1 * (cov @ w1), axis=0, keepdims=True)
    scale = g1 / jnp.sqrt(var1 + 1e-5)
    w1p = w1 * scale
    b1p = (b1 - mu1) * scale + be1
    return w1p, b1p


def _fold_bn2(s128, ss128, gn, bn):
    s = s128.reshape(4, HID).sum(axis=0, keepdims=True)
    ss = ss128.reshape(4, HID).sum(axis=0, keepdims=True)
    mu = s / N
    var = ss / N - mu * mu
    scale = gn / jnp.sqrt(var + 1e-5)
    bias = bn - mu * scale
    return scale, bias


def _blockdiag(w):
    return jnp.kron(jnp.eye(4, dtype=jnp.float32), w)


def kernel(x, t, edge_index, node_mask, edge_mask, ond, oed, params):
    layer_params, (wf, bf) = params
    src = edge_index[0]
    dst = edge_index[1]

    npad = EPAD - E
    qsrc = 2 * ((src % NR) * 4 + src // NR)
    qdst = (dst % NR) * 4 + dst // NR
    src2 = jnp.concatenate([qsrc, jnp.zeros((npad,), jnp.int32)]
                           ).reshape(-1, SUB)
    dst2 = jnp.concatenate([qdst, jnp.full((npad,), N, jnp.int32)]
                           ).reshape(-1, SUB)
    zrows = jnp.zeros((ZCH, 16), jnp.float32)

    xs4 = [x[g * NR:(g + 1) * NR] for g in range(4)]
    h = _build_pass(t.reshape(1, 1), xs4)

    acc = jnp.zeros((NR, 128), jnp.float32)
    for l, (eps, w1, b1, g1, be1, w2, b2, gn, bn) in enumerate(layer_params):
        if l == 0:
            w1 = jnp.concatenate([w1, jnp.zeros((HID - 13, HID), jnp.float32)], 0)
        a = (1.0 + eps).reshape(1, 1)
        sc_fn = _make_sc_scatter(1 if l == 0 else 2)
        agg = sc_fn(h.reshape(2 * N, 16), src2, dst2, zrows).reshape(NR, 128)
        sum128, m2128 = _stats_pass(a, h, agg, l1=(l == 0))
        w1p, b1p = _fold_bn1(sum128, m2128, w1, b1.reshape(1, -1),
                             g1.reshape(1, -1), be1.reshape(1, -1))
        y2, s128, ss128 = _mlp_pass(a, h, agg, _blockdiag(w1p),
                                    jnp.tile(b1p, (1, 4)), _blockdiag(w2),
                                    jnp.tile(b2.reshape(1, -1), (1, 4)),
                                    l1=(l == 0))
        scale2, bias2 = _fold_bn2(s128, ss128, gn.reshape(1, -1),
                                  bn.reshape(1, -1))
        wfp = jnp.pad(wf[l * HID:(l + 1) * HID], ((0, 0), (0, 20)))
        h, acc = _norm_pass(y2, jnp.tile(scale2, (1, 4)),
                            jnp.tile(bias2, (1, 4)), _blockdiag(wfp), acc)

    mcode = (node_mask.astype(jnp.int32)
             + 2 * edge_mask.astype(jnp.int32)).reshape(N, 1)
    ms4 = [mcode[g * NR:(g + 1) * NR] for g in range(4)]
    ondp = jnp.asarray(ond, jnp.int32).reshape(1, 1)
    oedp = jnp.asarray(oed, jnp.int32).reshape(1, 1)
    return _final_pass(ondp, oedp, bf.reshape(1, -1), acc, xs4, ms4)
